# Initial kernel scaffold; baseline (speedup 1.0000x reference)
import jax
import jax.numpy as jnp
from jax.experimental import pallas as pl


def kernel(x, play_emb, hand_emb, W1, b1, W2, b2):
    raise NotImplementedError("write your pallas kernel here")


# speedup vs baseline: 25.4286x; 25.4286x over previous
import jax
import jax.numpy as jnp
from jax import lax
from jax.experimental import pallas as pl


def _body(x_ref, pe_ref, he_ref, w1_ref, b1_ref, w2_ref, b2_ref, o_ref):
    bt = x_ref.shape[0]
    xb = x_ref[...]
    a1 = xb[:, 0:56]
    a2 = xb[:, 56:112]
    a3 = xb[:, 112:168]
    a4 = xb[:, 168:222]
    a4p = jnp.concatenate([a4, jnp.full((bt, 2), -1, jnp.int32)], axis=1)

    lane = lax.broadcasted_iota(jnp.int32, (bt, 56, 128), 2)
    cls = lane & 63
    sel = lane < 64
    idx12 = jnp.where(sel, a1[:, :, None], a2[:, :, None])
    cnt12 = jnp.sum((idx12 == cls).astype(jnp.float32), axis=1)
    idx34 = jnp.where(sel, a3[:, :, None], a4p[:, :, None])
    cnt34 = jnp.sum((idx34 == cls).astype(jnp.float32), axis=1)

    pe = pe_ref[...]
    he = he_ref[...]
    w1 = w1_ref[...]
    nt = (((1,), (1,)), ((), ()))
    f32 = jnp.float32
    A1 = lax.dot_general(pe, w1[:, 0:12], nt, preferred_element_type=f32)
    A2 = lax.dot_general(pe, w1[:, 12:24], nt, preferred_element_type=f32)
    A3 = lax.dot_general(pe, w1[:, 24:36], nt, preferred_element_type=f32)
    A4 = lax.dot_general(he, w1[:, 36:56], nt, preferred_element_type=f32)

    dot = lambda c, A: lax.dot_general(c, A, (((1,), (0,)), ((), ())),
                                       preferred_element_type=f32)
    hp = (dot(cnt12[:, 0:56], A1) + dot(cnt12[:, 64:120], A2)
          + dot(cnt34[:, 0:56], A3) + dot(cnt34[:, 64:118], A4)
          + b1_ref[...])
    h = jnp.maximum(hp, 0.0)
    out = lax.dot_general(h, w2_ref[...], nt, preferred_element_type=f32)
    out = out + b2_ref[...]
    skip = jnp.concatenate(
        [a4.astype(jnp.float32), jnp.zeros((bt, 1), jnp.float32)], axis=1)
    o_ref[...] = out + skip


def kernel(x, play_emb, hand_emb, W1, b1, W2, b2):
    B = x.shape[0]
    BT = 128 if B % 128 == 0 else B
    grid = (B // BT,)
    out = pl.pallas_call(
        _body,
        grid=grid,
        in_specs=[
            pl.BlockSpec((BT, 222), lambda i: (i, 0)),
            pl.BlockSpec((56, 12), lambda i: (0, 0)),
            pl.BlockSpec((54, 20), lambda i: (0, 0)),
            pl.BlockSpec((128, 56), lambda i: (0, 0)),
            pl.BlockSpec((1, 128), lambda i: (0, 0)),
            pl.BlockSpec((55, 128), lambda i: (0, 0)),
            pl.BlockSpec((1, 55), lambda i: (0, 0)),
        ],
        out_specs=pl.BlockSpec((BT, 55), lambda i: (i, 0)),
        out_shape=jax.ShapeDtypeStruct((B, 55), jnp.float32),
    )(x, play_emb, hand_emb, W1, b1.reshape(1, 128), W2, b2.reshape(1, 55))
    return out


# speedup vs baseline: 51.4037x; 2.0215x over previous
import functools

import jax
import jax.numpy as jnp
from jax import lax
from jax.experimental import pallas as pl
from jax.experimental.pallas import tpu as pltpu
from jax.experimental.pallas import tpu_sc as plsc

_ROWS = 280
_SUB = 128


def _sc_histogram(x):
    B, C = x.shape
    info = plsc.get_sparse_core_info()
    nw = info.num_cores * info.num_subcores
    per_w = B // nw
    n_sub = per_w // _SUB
    mesh = plsc.VectorSubcoreMesh(core_axis_name="c", subcore_axis_name="s")

    @functools.partial(
        pl.kernel,
        out_type=jax.ShapeDtypeStruct((_ROWS, B), jnp.float32),
        mesh=mesh,
        scratch_types=[
            pltpu.VMEM((_SUB, C), jnp.int32),
            pltpu.VMEM((_ROWS, _SUB), jnp.float32),
        ],
        compiler_params=pltpu.CompilerParams(use_tc_tiling_on_sc=False,
                                             needs_layout_passes=False),
    )
    def k(x_hbm, out_hbm, xs_v, cnt_v):
        wid = lax.axis_index("s") * info.num_cores + lax.axis_index("c")
        lanes = lax.iota(jnp.int32, 16)
        ones = jnp.full((16,), 1.0, jnp.float32)
        zeros16 = jnp.zeros((16,), jnp.float32)

        def subchunk(sc_i, carry):
            row0 = pl.multiple_of(wid * per_w + sc_i * _SUB, _SUB)
            pltpu.sync_copy(x_hbm.at[pl.ds(row0, _SUB), :], xs_v)

            def zero_body(r, c):
                for cg in range(_SUB // 16):
                    cnt_v[r, pl.ds(cg * 16, 16)] = zeros16
                return c
            lax.fori_loop(0, _ROWS, zero_body, 0)

            for g in range(_SUB // 16):
                rows = g * 16 + lanes

                def seg_body(j, c, boff, with_skip):
                    jv = jnp.full((16,), j, jnp.int32)
                    idx = plsc.load_gather(xs_v, [rows, jv])
                    plsc.addupdate_scatter(cnt_v, [idx + boff, rows], ones)
                    if with_skip:
                        plsc.store_scatter(cnt_v, [jv + (224 - 168), rows],
                                           idx.astype(jnp.float32))
                    return c

                for seg in range(4):
                    lo = 56 * seg
                    hi = min(56 * seg + 56, C)
                    lax.fori_loop(
                        lo, hi,
                        functools.partial(seg_body, boff=56 * seg,
                                          with_skip=(seg == 3)),
                        0)
            pltpu.sync_copy(cnt_v, out_hbm.at[:, pl.ds(row0, _SUB)])
            return carry

        lax.fori_loop(0, n_sub, subchunk, 0)

    return k(x)


def _mlp_body(s_ref, pe_ref, he_ref, w1_ref, b1_ref, w2_ref, b2_ref, o_ref):
    pe = pe_ref[...]
    he = he_ref[...]
    w1 = w1_ref[...]
    f32 = jnp.float32
    tt = (((1,), (1,)), ((), ()))
    nn = (((1,), (0,)), ((), ()))
    A1T = lax.dot_general(w1[:, 0:12], pe, tt, preferred_element_type=f32)
    A2T = lax.dot_general(w1[:, 12:24], pe, tt, preferred_element_type=f32)
    A3T = lax.dot_general(w1[:, 24:36], pe, tt, preferred_element_type=f32)
    A4T = lax.dot_general(w1[:, 36:56], he, tt, preferred_element_type=f32)
    s = s_ref[...]
    hp = (lax.dot_general(A1T, s[0:56], nn, preferred_element_type=f32)
          + lax.dot_general(A2T, s[56:112], nn, preferred_element_type=f32)
          + lax.dot_general(A3T, s[112:168], nn, preferred_element_type=f32)
          + lax.dot_general(A4T, s[168:224], nn, preferred_element_type=f32)
          + b1_ref[...])
    h = jnp.maximum(hp, 0.0)
    out = lax.dot_general(w2_ref[...], h, nn, preferred_element_type=f32)
    o_ref[...] = out + b2_ref[...] + s[224:280]


def _tc_mlp(S, play_emb, hand_emb, W1, b1, W2, b2):
    B = S.shape[1]
    BT = 512
    he56 = jnp.concatenate([hand_emb, jnp.zeros((2, 20), jnp.float32)], axis=0)
    w2p = jnp.concatenate([W2, jnp.zeros((1, 128), jnp.float32)], axis=0)
    b2p = jnp.concatenate([b2, jnp.zeros((1,), jnp.float32)]).reshape(56, 1)
    return pl.pallas_call(
        _mlp_body,
        grid=(B // BT,),
        in_specs=[
            pl.BlockSpec((_ROWS, BT), lambda i: (0, i)),
            pl.BlockSpec((56, 12), lambda i: (0, 0)),
            pl.BlockSpec((56, 20), lambda i: (0, 0)),
            pl.BlockSpec((128, 56), lambda i: (0, 0)),
            pl.BlockSpec((128, 1), lambda i: (0, 0)),
            pl.BlockSpec((56, 128), lambda i: (0, 0)),
            pl.BlockSpec((56, 1), lambda i: (0, 0)),
        ],
        out_specs=pl.BlockSpec((56, BT), lambda i: (0, i)),
        out_shape=jax.ShapeDtypeStruct((56, B), jnp.float32),
    )(S, play_emb, he56, W1, b1.reshape(128, 1), w2p, b2p)


def kernel(x, play_emb, hand_emb, W1, b1, W2, b2):
    S = _sc_histogram(x)
    out_t = _tc_mlp(S, play_emb, hand_emb, W1, b1, W2, b2)
    return out_t.T[:, :55]


# speedup vs baseline: 55.2174x; 1.0742x over previous
import functools

import jax
import jax.numpy as jnp
from jax import lax
from jax.experimental import pallas as pl
from jax.experimental.pallas import tpu as pltpu
from jax.experimental.pallas import tpu_sc as plsc

_ROWS = 280
_SUB = 128


def _sc_histogram(x):
    B, C = x.shape
    info = plsc.get_sparse_core_info()
    nw = info.num_cores * info.num_subcores
    per_w = B // nw
    n_sub = per_w // _SUB
    mesh = plsc.VectorSubcoreMesh(core_axis_name="c", subcore_axis_name="s")

    @functools.partial(
        pl.kernel,
        out_type=jax.ShapeDtypeStruct((_ROWS, B), jnp.float32),
        mesh=mesh,
        scratch_types=[
            pltpu.VMEM((_SUB, C), jnp.int32),
            pltpu.VMEM((_ROWS, _SUB), jnp.float32),
            pltpu.VMEM((_ROWS, _SUB), jnp.float32),
            pltpu.SemaphoreType.DMA,
            pltpu.SemaphoreType.DMA,
        ],
        compiler_params=pltpu.CompilerParams(use_tc_tiling_on_sc=False,
                                             needs_layout_passes=False),
    )
    def k(x_hbm, out_hbm, xs_v, cnt_a, cnt_b, sem_a, sem_b):
        wid = lax.axis_index("s") * info.num_cores + lax.axis_index("c")
        lanes = lax.iota(jnp.int32, 16)
        ones = jnp.full((16,), 1.0, jnp.float32)
        zeros16 = jnp.zeros((16,), jnp.float32)
        rows_list = [g * 16 + lanes for g in range(_SUB // 16)]
        bufs = (cnt_a, cnt_b)
        sems = (sem_a, sem_b)
        pending = [None, None]

        for cnt in bufs:
            for r in (278, 279):
                for cg in range(_SUB // 16):
                    cnt[r, pl.ds(cg * 16, 16)] = zeros16

        for sc_i in range(n_sub):
            cnt, sem = bufs[sc_i % 2], sems[sc_i % 2]
            row0 = pl.multiple_of(wid * per_w + sc_i * _SUB, _SUB)
            pltpu.sync_copy(x_hbm.at[pl.ds(row0, _SUB), :], xs_v)
            if pending[sc_i % 2] is not None:
                pending[sc_i % 2].wait()

            def zero_body(r, c, cnt=cnt):
                for cg in range(_SUB // 16):
                    cnt[r, pl.ds(cg * 16, 16)] = zeros16
                return c
            lax.fori_loop(0, 224, zero_body, 0)

            for seg in range(4):
                boff = 56 * seg
                hi = min(boff + 56, C)

                def seg_body(j, c, cnt=cnt, boff=boff, skip=(seg == 3)):
                    jv = jnp.full((16,), j, jnp.int32)
                    for rows in rows_list:
                        idx = plsc.load_gather(xs_v, [rows, jv])
                        plsc.addupdate_scatter(cnt, [idx + boff, rows], ones)
                        if skip:
                            plsc.store_scatter(cnt, [jv + (224 - 168), rows],
                                               idx.astype(jnp.float32))
                    return c
                lax.fori_loop(boff, hi, seg_body, 0)

            pending[sc_i % 2] = pltpu.async_copy(
                cnt, out_hbm.at[:, pl.ds(row0, _SUB)], sem)
        for p in pending:
            if p is not None:
                p.wait()

    return k(x)


def _mlp_body(s_ref, pe_ref, he_ref, w1_ref, b1_ref, w2_ref, b2_ref,
              eye_ref, o_ref):
    pe = pe_ref[...]
    he = he_ref[...]
    w1 = w1_ref[...]
    f32 = jnp.float32
    tt = (((1,), (1,)), ((), ()))
    nn = (((1,), (0,)), ((), ()))
    lt = (((0,), (0,)), ((), ()))
    A1T = lax.dot_general(w1[:, 0:12], pe, tt, preferred_element_type=f32)
    A2T = lax.dot_general(w1[:, 12:24], pe, tt, preferred_element_type=f32)
    A3T = lax.dot_general(w1[:, 24:36], pe, tt, preferred_element_type=f32)
    A4T = lax.dot_general(w1[:, 36:56], he, tt, preferred_element_type=f32)
    s = s_ref[...]
    hp = (lax.dot_general(A1T, s[0:56], nn, preferred_element_type=f32)
          + lax.dot_general(A2T, s[56:112], nn, preferred_element_type=f32)
          + lax.dot_general(A3T, s[112:168], nn, preferred_element_type=f32)
          + lax.dot_general(A4T, s[168:224], nn, preferred_element_type=f32)
          + b1_ref[...])
    h = jnp.maximum(hp, 0.0)
    out = lax.dot_general(h, w2_ref[...], (((0,), (1,)), ((), ())),
                          preferred_element_type=f32)
    skip = lax.dot_general(s[224:280], eye_ref[...], lt,
                           preferred_element_type=f32)
    o_ref[...] = out + skip + b2_ref[...]


def _tc_mlp(S, play_emb, hand_emb, W1, b1, W2, b2):
    B = S.shape[1]
    BT = 512
    he56 = jnp.concatenate([hand_emb, jnp.zeros((2, 20), jnp.float32)], axis=0)
    eye = jnp.eye(56, 55, dtype=jnp.float32)
    return pl.pallas_call(
        _mlp_body,
        grid=(B // BT,),
        in_specs=[
            pl.BlockSpec((_ROWS, BT), lambda i: (0, i)),
            pl.BlockSpec((56, 12), lambda i: (0, 0)),
            pl.BlockSpec((56, 20), lambda i: (0, 0)),
            pl.BlockSpec((128, 56), lambda i: (0, 0)),
            pl.BlockSpec((128, 1), lambda i: (0, 0)),
            pl.BlockSpec((55, 128), lambda i: (0, 0)),
            pl.BlockSpec((1, 55), lambda i: (0, 0)),
            pl.BlockSpec((56, 55), lambda i: (0, 0)),
        ],
        out_specs=pl.BlockSpec((BT, 55), lambda i: (i, 0)),
        out_shape=jax.ShapeDtypeStruct((B, 55), jnp.float32),
    )(S, play_emb, he56, W1, b1.reshape(128, 1), W2, b2.reshape(1, 55), eye)


def kernel(x, play_emb, hand_emb, W1, b1, W2, b2):
    S = _sc_histogram(x)
    return _tc_mlp(S, play_emb, hand_emb, W1, b1, W2, b2)


# speedup vs baseline: 72.0140x; 1.3042x over previous
import functools

import jax
import jax.numpy as jnp
from jax import lax
from jax.experimental import pallas as pl
from jax.experimental.pallas import tpu as pltpu
from jax.experimental.pallas import tpu_sc as plsc

_ROWS = 280
_SUB = 128


def _sc_histogram(x):
    B, C = x.shape
    info = plsc.get_sparse_core_info()
    nw = info.num_cores * info.num_subcores
    per_w = B // nw
    n_sub = per_w // _SUB
    mesh = plsc.VectorSubcoreMesh(core_axis_name="c", subcore_axis_name="s")

    @functools.partial(
        pl.kernel,
        out_type=jax.ShapeDtypeStruct((_ROWS, B), jnp.float32),
        mesh=mesh,
        scratch_types=[
            pltpu.VMEM((64, C), jnp.int32),
            pltpu.VMEM((64, C), jnp.int32),
            pltpu.VMEM((_ROWS, _SUB), jnp.float32),
            pltpu.VMEM((_ROWS, _SUB), jnp.float32),
            pltpu.SemaphoreType.DMA,
            pltpu.SemaphoreType.DMA,
            pltpu.SemaphoreType.DMA,
            pltpu.SemaphoreType.DMA,
        ],
        compiler_params=pltpu.CompilerParams(use_tc_tiling_on_sc=False,
                                             needs_layout_passes=False),
    )
    def k(x_hbm, out_hbm, xs_a, xs_b, cnt_a, cnt_b,
          isem_a, isem_b, osem_a, osem_b):
        wid = lax.axis_index("s") * info.num_cores + lax.axis_index("c")
        lanes = lax.iota(jnp.int32, 16)
        ones = jnp.full((16,), 1.0, jnp.float32)
        zeros16 = jnp.zeros((16,), jnp.float32)
        xbufs, isems = (xs_a, xs_b), (isem_a, isem_b)
        cbufs, osems = (cnt_a, cnt_b), (osem_a, osem_b)
        pend_in = [None, None]
        pend_out = [None, None]
        n_half = per_w // 64

        def x_copy(hidx, buf, sem):
            row0 = pl.multiple_of(wid * per_w + hidx * 64, 64)
            return pltpu.async_copy(x_hbm.at[pl.ds(row0, 64), :], buf, sem)

        pend_in[0] = x_copy(0, xbufs[0], isems[0])

        for cnt in cbufs:
            for r in (278, 279):
                for cg in range(_SUB // 16):
                    cnt[r, pl.ds(cg * 16, 16)] = zeros16

        for sc_i in range(n_sub):
            cnt, osem = cbufs[sc_i % 2], osems[sc_i % 2]
            row0 = pl.multiple_of(wid * per_w + sc_i * _SUB, _SUB)
            if pend_out[sc_i % 2] is not None:
                pend_out[sc_i % 2].wait()

            @plsc.parallel_loop(0, 224, unroll=4)
            def _(r, cnt=cnt):
                for cg in range(_SUB // 16):
                    cnt[r, pl.ds(cg * 16, 16)] = zeros16

            for half in range(2):
                hidx = sc_i * 2 + half
                xs = xbufs[hidx % 2]
                pend_in[hidx % 2].wait()
                if hidx + 1 < n_half:
                    nb = (hidx + 1) % 2
                    pend_in[nb] = x_copy(hidx + 1, xbufs[nb], isems[nb])
                groups = [(g * 16 + lanes, half * 64 + g * 16 + lanes)
                          for g in range(4)]
                for seg in range(4):
                    boff = 56 * seg
                    hi = min(boff + 56, C)
                    cnt_seg = cnt.at[pl.ds(boff, 56), :]

                    @plsc.parallel_loop(boff, hi, unroll=2)
                    def _(j, xs=xs, cnt=cnt, cnt_seg=cnt_seg,
                          skip=(seg == 3), groups=groups):
                        jv = jnp.full((16,), j, jnp.int32)
                        idxs = [plsc.load_gather(xs, [rows, jv])
                                for rows, _ in groups]
                        for (rows, cols), idx in zip(groups, idxs):
                            plsc.addupdate_scatter(cnt_seg, [idx, cols], ones)
                        if skip:
                            for (rows, cols), idx in zip(groups, idxs):
                                plsc.store_scatter(cnt, [jv + 56, cols],
                                                   idx.astype(jnp.float32))

            pend_out[sc_i % 2] = pltpu.async_copy(
                cnt, out_hbm.at[:, pl.ds(row0, _SUB)], osem)
        for p in pend_out:
            if p is not None:
                p.wait()

    return k(x)


def _mlp_body(s_ref, pe_ref, he_ref, w1_ref, b1_ref, w2_ref, b2_ref,
              eye_ref, o_ref):
    pe = pe_ref[...]
    he = he_ref[...]
    w1 = w1_ref[...]
    f32 = jnp.float32
    tt = (((1,), (1,)), ((), ()))
    nn = (((1,), (0,)), ((), ()))
    lt = (((0,), (0,)), ((), ()))
    A1T = lax.dot_general(w1[:, 0:12], pe, tt, preferred_element_type=f32)
    A2T = lax.dot_general(w1[:, 12:24], pe, tt, preferred_element_type=f32)
    A3T = lax.dot_general(w1[:, 24:36], pe, tt, preferred_element_type=f32)
    A4T = lax.dot_general(w1[:, 36:56], he, tt, preferred_element_type=f32)
    s = s_ref[...]
    hp = (lax.dot_general(A1T, s[0:56], nn, preferred_element_type=f32)
          + lax.dot_general(A2T, s[56:112], nn, preferred_element_type=f32)
          + lax.dot_general(A3T, s[112:168], nn, preferred_element_type=f32)
          + lax.dot_general(A4T, s[168:224], nn, preferred_element_type=f32)
          + b1_ref[...])
    h = jnp.maximum(hp, 0.0)
    out = lax.dot_general(h, w2_ref[...], (((0,), (1,)), ((), ())),
                          preferred_element_type=f32)
    skip = lax.dot_general(s[224:280], eye_ref[...], lt,
                           preferred_element_type=f32)
    o_ref[...] = out + skip + b2_ref[...]


def _tc_mlp(S, play_emb, hand_emb, W1, b1, W2, b2):
    B = S.shape[1]
    BT = 512
    he56 = jnp.concatenate([hand_emb, jnp.zeros((2, 20), jnp.float32)], axis=0)
    eye = jnp.eye(56, 55, dtype=jnp.float32)
    return pl.pallas_call(
        _mlp_body,
        grid=(B // BT,),
        in_specs=[
            pl.BlockSpec((_ROWS, BT), lambda i: (0, i)),
            pl.BlockSpec((56, 12), lambda i: (0, 0)),
            pl.BlockSpec((56, 20), lambda i: (0, 0)),
            pl.BlockSpec((128, 56), lambda i: (0, 0)),
            pl.BlockSpec((128, 1), lambda i: (0, 0)),
            pl.BlockSpec((55, 128), lambda i: (0, 0)),
            pl.BlockSpec((1, 55), lambda i: (0, 0)),
            pl.BlockSpec((56, 55), lambda i: (0, 0)),
        ],
        out_specs=pl.BlockSpec((BT, 55), lambda i: (i, 0)),
        out_shape=jax.ShapeDtypeStruct((B, 55), jnp.float32),
    )(S, play_emb, he56, W1, b1.reshape(128, 1), W2, b2.reshape(1, 55), eye)


def kernel(x, play_emb, hand_emb, W1, b1, W2, b2):
    S = _sc_histogram(x)
    return _tc_mlp(S, play_emb, hand_emb, W1, b1, W2, b2)
